# baseline (speedup 1.0000x reference)
import math

import jax
import jax.numpy as jnp
from jax import lax
from jax.experimental import pallas as pl
from jax.experimental.pallas import tpu as pltpu

_THRESH = 0.8
_MIN_KEPT = 100000
_ROWS = 32
_SUB = 8
_L08 = -math.log(_THRESH)


def _make_fused_body(write_loss):
    def body(lthr_ref, score_ref, tgt_ref, *out_refs):
        sum_ref, cnt_ref = out_refs[0], out_refs[1]
        i = pl.program_id(0)
        j = pl.program_id(1)
        c = score_ref.shape[1]
        w = score_ref.shape[3]
        thr = lthr_ref[i]
        acc = jnp.zeros((_SUB, w), jnp.float32)
        cnt = jnp.zeros((_SUB, w), jnp.float32)
        for rt in range(_ROWS // _SUB):
            rows = pl.ds(rt * _SUB, _SUB)
            t = tgt_ref[0, rows, :]
            m = score_ref[0, 0, rows, :]
            for cc in range(1, c):
                m = jnp.maximum(m, score_ref[0, cc, rows, :])
            s = jnp.zeros((_SUB, w), jnp.float32)
            xt = jnp.zeros((_SUB, w), jnp.float32)
            for cc in range(c):
                d = score_ref[0, cc, rows, :] - m
                s = s + jnp.exp(d)
                xt = xt + jnp.where(t == cc, d, 0.0)
            loss = jnp.log(s) - xt
            if write_loss:
                out_refs[2][0, rows, :] = loss
            keep = loss > thr
            acc = acc + jnp.where(keep, loss, 0.0)
            cnt = cnt + keep.astype(jnp.float32)

        @pl.when(j == 0)
        def _():
            sum_ref[i] = 0.0
            cnt_ref[i] = 0.0

        sum_ref[i] += jnp.sum(acc)
        cnt_ref[i] += jnp.sum(cnt)

    return body


def _select_body(k_sel_ref, loss_ref, lk_ref):
    i = pl.program_id(0)
    k_sel = k_sel_ref[0].astype(jnp.float32)

    def step(idx, lo):
        mid = lo + jnp.left_shift(jnp.int32(1), 30 - idx)
        bits = lax.bitcast_convert_type(loss_ref[...], jnp.int32)
        cnt = jnp.sum((bits < mid).astype(jnp.float32))
        return jnp.where(cnt < k_sel, mid, lo)

    t_star = lax.fori_loop(0, 31, step, jnp.int32(0))
    lk_ref[i] = lax.bitcast_convert_type(t_star, jnp.float32)


@jax.jit
def kernel(score, target):
    b, c, h, w = score.shape
    target = target.astype(jnp.int32)
    nblk = h // _ROWS
    k0 = min(_MIN_KEPT, h * w - 1)

    def fused_call(write_loss):
        out_shape = [
            jax.ShapeDtypeStruct((b,), jnp.float32),
            jax.ShapeDtypeStruct((b,), jnp.float32),
        ]
        out_specs = [
            pl.BlockSpec(memory_space=pltpu.SMEM),
            pl.BlockSpec(memory_space=pltpu.SMEM),
        ]
        if write_loss:
            out_shape.append(jax.ShapeDtypeStruct((b, h, w), jnp.float32))
            out_specs.append(pl.BlockSpec((1, _ROWS, w), lambda i, j: (i, j, 0)))
        return pl.pallas_call(
            _make_fused_body(write_loss),
            grid=(b, nblk),
            in_specs=[
                pl.BlockSpec(memory_space=pltpu.SMEM),
                pl.BlockSpec((1, c, _ROWS, w), lambda i, j: (i, 0, j, 0)),
                pl.BlockSpec((1, _ROWS, w), lambda i, j: (i, j, 0)),
            ],
            out_specs=out_specs,
            out_shape=out_shape,
        )

    select = pl.pallas_call(
        _select_body,
        grid=(b,),
        in_specs=[
            pl.BlockSpec(memory_space=pltpu.SMEM),
            pl.BlockSpec((1, h, w), lambda i: (i, 0, 0)),
        ],
        out_specs=pl.BlockSpec(memory_space=pltpu.SMEM),
        out_shape=jax.ShapeDtypeStruct((b,), jnp.float32),
    )

    thr0 = jnp.full((b,), _L08, jnp.float32)
    sums, cnts = fused_call(False)(thr0, score, target)

    def rare_path(_):
        _, _, loss = fused_call(True)(thr0, score, target)
        lk = select(jnp.array([h * w - k0], jnp.int32), loss)
        s2, c2 = fused_call(False)(jnp.minimum(lk, _L08), score, target)
        return s2, c2

    sums, cnts = lax.cond(jnp.any(cnts < float(k0 + 1)), rare_path,
                          lambda _: (sums, cnts), operand=None)
    return jnp.sum(sums / jnp.maximum(cnts, 1.0)) / b


# speedup vs baseline: 1.6760x; 1.6760x over previous
import math

import jax
import jax.numpy as jnp
from jax import lax
from jax.experimental import pallas as pl
from jax.experimental.pallas import tpu as pltpu

_THRESH = 0.8
_MIN_KEPT = 100000
_ROWS = 128
_SUB = 8
_L08 = -math.log(_THRESH)


def _make_fused_body(write_loss):
    def body(lthr_ref, score_ref, tgt_ref, *out_refs):
        sum_ref, cnt_ref = out_refs[0], out_refs[1]
        i = pl.program_id(0)
        j = pl.program_id(1)
        c = score_ref.shape[1]
        w = score_ref.shape[3]
        thr = lthr_ref[i]
        acc = jnp.zeros((_SUB, w), jnp.float32)
        cnt = jnp.zeros((_SUB, w), jnp.float32)
        for rt in range(_ROWS // _SUB):
            rows = pl.ds(rt * _SUB, _SUB)
            t = tgt_ref[0, rows, :]
            m = score_ref[0, 0, rows, :]
            for cc in range(1, c):
                m = jnp.maximum(m, score_ref[0, cc, rows, :])
            s = jnp.zeros((_SUB, w), jnp.float32)
            xt = jnp.zeros((_SUB, w), jnp.float32)
            for cc in range(c):
                d = score_ref[0, cc, rows, :] - m
                s = s + jnp.exp(d)
                xt = xt + jnp.where(t == cc, d, 0.0)
            loss = jnp.log(s) - xt
            if write_loss:
                out_refs[2][0, rows, :] = loss
            keep = loss > thr
            acc = acc + jnp.where(keep, loss, 0.0)
            cnt = cnt + keep.astype(jnp.float32)

        @pl.when(j == 0)
        def _():
            sum_ref[i] = 0.0
            cnt_ref[i] = 0.0

        sum_ref[i] += jnp.sum(acc)
        cnt_ref[i] += jnp.sum(cnt)

    return body


def _select_body(k_sel_ref, loss_ref, lk_ref):
    i = pl.program_id(0)
    k_sel = k_sel_ref[0].astype(jnp.float32)

    def step(idx, lo):
        mid = lo + jnp.left_shift(jnp.int32(1), 30 - idx)
        bits = lax.bitcast_convert_type(loss_ref[...], jnp.int32)
        cnt = jnp.sum((bits < mid).astype(jnp.float32))
        return jnp.where(cnt < k_sel, mid, lo)

    t_star = lax.fori_loop(0, 31, step, jnp.int32(0))
    lk_ref[i] = lax.bitcast_convert_type(t_star, jnp.float32)


@jax.jit
def kernel(score, target):
    b, c, h, w = score.shape
    target = target.astype(jnp.int32)
    nblk = h // _ROWS
    k0 = min(_MIN_KEPT, h * w - 1)

    def fused_call(write_loss):
        out_shape = [
            jax.ShapeDtypeStruct((b,), jnp.float32),
            jax.ShapeDtypeStruct((b,), jnp.float32),
        ]
        out_specs = [
            pl.BlockSpec(memory_space=pltpu.SMEM),
            pl.BlockSpec(memory_space=pltpu.SMEM),
        ]
        if write_loss:
            out_shape.append(jax.ShapeDtypeStruct((b, h, w), jnp.float32))
            out_specs.append(pl.BlockSpec((1, _ROWS, w), lambda i, j: (i, j, 0)))
        return pl.pallas_call(
            _make_fused_body(write_loss),
            grid=(b, nblk),
            in_specs=[
                pl.BlockSpec(memory_space=pltpu.SMEM),
                pl.BlockSpec((1, c, _ROWS, w), lambda i, j: (i, 0, j, 0)),
                pl.BlockSpec((1, _ROWS, w), lambda i, j: (i, j, 0)),
            ],
            out_specs=out_specs,
            out_shape=out_shape,
        )

    select = pl.pallas_call(
        _select_body,
        grid=(b,),
        in_specs=[
            pl.BlockSpec(memory_space=pltpu.SMEM),
            pl.BlockSpec((1, h, w), lambda i: (i, 0, 0)),
        ],
        out_specs=pl.BlockSpec(memory_space=pltpu.SMEM),
        out_shape=jax.ShapeDtypeStruct((b,), jnp.float32),
    )

    thr0 = jnp.full((b,), _L08, jnp.float32)
    sums, cnts = fused_call(False)(thr0, score, target)

    def rare_path(_):
        _, _, loss = fused_call(True)(thr0, score, target)
        lk = select(jnp.array([h * w - k0], jnp.int32), loss)
        s2, c2 = fused_call(False)(jnp.minimum(lk, _L08), score, target)
        return s2, c2

    sums, cnts = lax.cond(jnp.any(cnts < float(k0 + 1)), rare_path,
                          lambda _: (sums, cnts), operand=None)
    return jnp.sum(sums / jnp.maximum(cnts, 1.0)) / b


# speedup vs baseline: 1.8196x; 1.0857x over previous
import math

import jax
import jax.numpy as jnp
from jax import lax
from jax.experimental import pallas as pl
from jax.experimental.pallas import tpu as pltpu

_THRESH = 0.8
_MIN_KEPT = 100000
_ROWS = 256
_SUB = 8
_L08 = -math.log(_THRESH)


def _make_fused_body(write_loss):
    def body(lthr_ref, score_ref, tgt_ref, *out_refs):
        sum_ref, cnt_ref = out_refs[0], out_refs[1]
        i = pl.program_id(0)
        j = pl.program_id(1)
        c = score_ref.shape[1]
        w = score_ref.shape[3]
        thr = lthr_ref[i]
        acc = jnp.zeros((_SUB, w), jnp.float32)
        cnt = jnp.zeros((_SUB, w), jnp.float32)
        for rt in range(_ROWS // _SUB):
            rows = pl.ds(rt * _SUB, _SUB)
            t = tgt_ref[0, rows, :]
            m = score_ref[0, 0, rows, :]
            for cc in range(1, c):
                m = jnp.maximum(m, score_ref[0, cc, rows, :])
            s = jnp.zeros((_SUB, w), jnp.float32)
            xt = jnp.zeros((_SUB, w), jnp.float32)
            for cc in range(c):
                d = score_ref[0, cc, rows, :] - m
                s = s + jnp.exp(d)
                xt = xt + jnp.where(t == cc, d, 0.0)
            loss = jnp.log(s) - xt
            if write_loss:
                out_refs[2][0, rows, :] = loss
            keep = loss > thr
            acc = acc + jnp.where(keep, loss, 0.0)
            cnt = cnt + keep.astype(jnp.float32)

        @pl.when(j == 0)
        def _():
            sum_ref[i] = 0.0
            cnt_ref[i] = 0.0

        sum_ref[i] += jnp.sum(acc)
        cnt_ref[i] += jnp.sum(cnt)

    return body


def _select_body(k_sel_ref, loss_ref, lk_ref):
    i = pl.program_id(0)
    k_sel = k_sel_ref[0].astype(jnp.float32)

    def step(idx, lo):
        mid = lo + jnp.left_shift(jnp.int32(1), 30 - idx)
        bits = lax.bitcast_convert_type(loss_ref[...], jnp.int32)
        cnt = jnp.sum((bits < mid).astype(jnp.float32))
        return jnp.where(cnt < k_sel, mid, lo)

    t_star = lax.fori_loop(0, 31, step, jnp.int32(0))
    lk_ref[i] = lax.bitcast_convert_type(t_star, jnp.float32)


@jax.jit
def kernel(score, target):
    b, c, h, w = score.shape
    target = target.astype(jnp.int32)
    nblk = h // _ROWS
    k0 = min(_MIN_KEPT, h * w - 1)

    def fused_call(write_loss):
        out_shape = [
            jax.ShapeDtypeStruct((b,), jnp.float32),
            jax.ShapeDtypeStruct((b,), jnp.float32),
        ]
        out_specs = [
            pl.BlockSpec(memory_space=pltpu.SMEM),
            pl.BlockSpec(memory_space=pltpu.SMEM),
        ]
        if write_loss:
            out_shape.append(jax.ShapeDtypeStruct((b, h, w), jnp.float32))
            out_specs.append(pl.BlockSpec((1, _ROWS, w), lambda i, j: (i, j, 0)))
        return pl.pallas_call(
            _make_fused_body(write_loss),
            grid=(b, nblk),
            in_specs=[
                pl.BlockSpec(memory_space=pltpu.SMEM),
                pl.BlockSpec((1, c, _ROWS, w), lambda i, j: (i, 0, j, 0)),
                pl.BlockSpec((1, _ROWS, w), lambda i, j: (i, j, 0)),
            ],
            out_specs=out_specs,
            out_shape=out_shape,
        )

    select = pl.pallas_call(
        _select_body,
        grid=(b,),
        in_specs=[
            pl.BlockSpec(memory_space=pltpu.SMEM),
            pl.BlockSpec((1, h, w), lambda i: (i, 0, 0)),
        ],
        out_specs=pl.BlockSpec(memory_space=pltpu.SMEM),
        out_shape=jax.ShapeDtypeStruct((b,), jnp.float32),
    )

    thr0 = jnp.full((b,), _L08, jnp.float32)
    sums, cnts = fused_call(False)(thr0, score, target)

    def rare_path(_):
        _, _, loss = fused_call(True)(thr0, score, target)
        lk = select(jnp.array([h * w - k0], jnp.int32), loss)
        s2, c2 = fused_call(False)(jnp.minimum(lk, _L08), score, target)
        return s2, c2

    sums, cnts = lax.cond(jnp.any(cnts < float(k0 + 1)), rare_path,
                          lambda _: (sums, cnts), operand=None)
    return jnp.sum(sums / jnp.maximum(cnts, 1.0)) / b
